# baseline (speedup 1.0000x reference)
import functools

import jax
import jax.numpy as jnp
from jax import lax
from jax.experimental import pallas as pl
from jax.experimental.pallas import tpu as pltpu
from jax.experimental.pallas import tpu_sc as plsc

BOOK = 8192
CODE = 64
BATCH = 8192
GAMMA = 0.99
ALPHA = 1.0 - GAMMA

NC, NS, L = 2, 16, 16
TPC = BATCH // NS
CPC = CODE // NC
CHUNK = 128
NCHUNK = TPC // CHUNK


def _body(x_hbm, idx_hbm, n_hbm, m_hbm, out_hbm,
          acc_sh, cnt_sh, idx_v, x_v, zb_v, ones_v, m_v, n_v, cnt_v, out_v,
          sem):
    c = lax.axis_index("c")
    s = lax.axis_index("s")
    t0 = s * TPC
    c0 = c * CPC

    zeros = jnp.zeros((L,), jnp.float32)
    ones = jnp.ones((L,), jnp.float32)

    def _fill(i, _):
        for k in range(4):
            r = i * 4 + k
            zb_v[r, pl.ds(0, L)] = zeros
            zb_v[r, pl.ds(L, L)] = zeros
            zb_v[CHUNK + r, pl.ds(0, L)] = zeros
            zb_v[CHUNK + r, pl.ds(L, L)] = zeros
            zb_v[2 * CHUNK + r, pl.ds(0, L)] = zeros
            zb_v[2 * CHUNK + r, pl.ds(L, L)] = zeros
            zb_v[3 * CHUNK + r, pl.ds(0, L)] = zeros
            zb_v[3 * CHUNK + r, pl.ds(L, L)] = zeros
            ones_v[r, pl.ds(0, L)] = ones
        return 0
    lax.fori_loop(0, CHUNK // 4, _fill, 0)

    pltpu.sync_copy(zb_v, acc_sh.at[pl.ds(t0, TPC), :])
    pltpu.sync_copy(zb_v.at[:, pl.ds(0, L)], cnt_sh.at[pl.ds(t0, TPC), :])
    pltpu.sync_copy(idx_hbm.at[pl.ds(s * NCHUNK, NCHUNK), :], idx_v)
    pltpu.sync_copy(x_hbm.at[pl.ds(t0, TPC), pl.ds(c0, CPC)], x_v)
    pltpu.sync_copy(m_hbm.at[pl.ds(t0, TPC), pl.ds(c0, CPC)], m_v)
    pltpu.sync_copy(n_hbm.at[pl.ds(t0, TPC), :], n_v)
    plsc.subcore_barrier()

    for j in range(NCHUNK):
        pltpu.sync_copy(x_v.at[pl.ds(j * CHUNK, CHUNK), :],
                        acc_sh.at[idx_v.at[j]], add=True)
        pltpu.sync_copy(ones_v, cnt_sh.at[idx_v.at[j]], add=True)
    plsc.subcore_barrier()

    pltpu.sync_copy(acc_sh.at[pl.ds(t0, TPC), :], x_v)
    pltpu.sync_copy(cnt_sh.at[pl.ds(t0, TPC), :], cnt_v)

    def _row(i, _):
        for k in range(4):
            r = i * 4 + k
            cvec = cnt_v[r, pl.ds(0, L)]
            nvec = n_v[r, pl.ds(0, L)]
            rv = 1.0 / (GAMMA * nvec + ALPHA * cvec)
            for g in range(CPC // L):
                mv = m_v[r, pl.ds(g * L, L)]
                av = x_v[r, pl.ds(g * L, L)]
                out_v[r, pl.ds(g * L, L)] = (GAMMA * mv + ALPHA * av) * rv
        return 0
    lax.fori_loop(0, TPC // 4, _row, 0)

    pltpu.sync_copy(out_v, out_hbm.at[pl.ds(t0, TPC), pl.ds(c0, CPC)])


_ema_update = pl.kernel(
    _body,
    out_type=jax.ShapeDtypeStruct((BOOK, CODE), jnp.float32),
    mesh=plsc.VectorSubcoreMesh(core_axis_name="c", subcore_axis_name="s",
                                num_cores=NC, num_subcores=NS),
    scratch_types=[
        pltpu.VMEM_SHARED((BOOK, CPC), jnp.float32),
        pltpu.VMEM_SHARED((BOOK, L), jnp.float32),
        pltpu.VMEM((NCHUNK, CHUNK), jnp.int32),
        pltpu.VMEM((TPC, CPC), jnp.float32),
        pltpu.VMEM((TPC, CPC), jnp.float32),
        pltpu.VMEM((CHUNK, L), jnp.float32),
        pltpu.VMEM((TPC, CPC), jnp.float32),
        pltpu.VMEM((TPC, L), jnp.float32),
        pltpu.VMEM((TPC, L), jnp.float32),
        pltpu.VMEM((TPC, CPC), jnp.float32),
        pltpu.SemaphoreType.DMA,
    ],
    compiler_params=pltpu.CompilerParams(use_tc_tiling_on_sc=False),
    name="vq_ema_update_sc",
)


@jax.jit
def kernel(inputs, distances, idx, N, m, codebook):
    del distances, codebook
    idx2 = idx.reshape(BATCH // CHUNK, CHUNK)
    n16 = jnp.broadcast_to(N, (BOOK, L))
    return _ema_update(inputs, idx2, n16, m)


# speedup vs baseline: 1.0737x; 1.0737x over previous
import functools

import jax
import jax.numpy as jnp
from jax import lax
from jax.experimental import pallas as pl
from jax.experimental.pallas import tpu as pltpu
from jax.experimental.pallas import tpu_sc as plsc

BOOK = 8192
CODE = 64
BATCH = 8192
GAMMA = 0.99
ALPHA = 1.0 - GAMMA

NC, NS, L = 2, 16, 16
TPC = BATCH // NS
CPC = CODE // NC
CHUNK = 128
NCHUNK = TPC // CHUNK


def _body(x_hbm, idx_hbm, n_hbm, m_hbm, out_hbm,
          acc_sh, cnt_sh, idx1_v, idx_v, x_v, zb_v, ones_v, m_v, n_v, cnt_v,
          out_v, sem, sem2):
    c = lax.axis_index("c")
    s = lax.axis_index("s")
    t0 = s * TPC
    c0 = c * CPC

    zeros = jnp.zeros((L,), jnp.float32)
    ones = jnp.ones((L,), jnp.float32)

    def _fill(i, _):
        for k in range(4):
            r = i * 4 + k
            zb_v[r, pl.ds(0, L)] = zeros
            zb_v[r, pl.ds(L, L)] = zeros
            zb_v[CHUNK + r, pl.ds(0, L)] = zeros
            zb_v[CHUNK + r, pl.ds(L, L)] = zeros
            zb_v[2 * CHUNK + r, pl.ds(0, L)] = zeros
            zb_v[2 * CHUNK + r, pl.ds(L, L)] = zeros
            zb_v[3 * CHUNK + r, pl.ds(0, L)] = zeros
            zb_v[3 * CHUNK + r, pl.ds(L, L)] = zeros
            ones_v[r, pl.ds(0, L)] = ones
        return 0
    lax.fori_loop(0, CHUNK // 4, _fill, 0)

    cps = [
        pltpu.async_copy(idx_hbm.at[pl.ds(t0, TPC)], idx1_v, sem),
        pltpu.async_copy(x_hbm.at[pl.ds(t0, TPC), pl.ds(c0, CPC)], x_v, sem),
    ]
    pltpu.sync_copy(zb_v, acc_sh.at[pl.ds(t0, TPC), :])
    pltpu.sync_copy(zb_v.at[:, pl.ds(0, L)], cnt_sh.at[pl.ds(t0, TPC), :])
    for cp in cps:
        cp.wait()
    for g in range(TPC // L):
        idx_v[g // (CHUNK // L), pl.ds((g % (CHUNK // L)) * L, L)] = (
            idx1_v[pl.ds(g * L, L)])
    plsc.subcore_barrier()

    for j in range(NCHUNK):
        pltpu.sync_copy(x_v.at[pl.ds(j * CHUNK, CHUNK), :],
                        acc_sh.at[idx_v.at[j]], add=True)
        pltpu.sync_copy(ones_v, cnt_sh.at[idx_v.at[j]], add=True)
    plsc.subcore_barrier()

    cp_m = pltpu.async_copy(m_hbm.at[pl.ds(t0, TPC), pl.ds(c0, CPC)], m_v,
                            sem2)
    cp_n = pltpu.async_copy(n_hbm.at[pl.ds(t0, TPC), :], n_v, sem2)
    pltpu.sync_copy(acc_sh.at[pl.ds(t0, TPC), :], x_v)
    pltpu.sync_copy(cnt_sh.at[pl.ds(t0, TPC), :], cnt_v)
    cp_m.wait()
    cp_n.wait()

    def _row(i, _):
        for k in range(4):
            r = i * 4 + k
            cvec = cnt_v[r, pl.ds(0, L)]
            nvec = n_v[r, pl.ds(0, L)]
            rv = 1.0 / (GAMMA * nvec + ALPHA * cvec)
            for g in range(CPC // L):
                mv = m_v[r, pl.ds(g * L, L)]
                av = x_v[r, pl.ds(g * L, L)]
                out_v[r, pl.ds(g * L, L)] = (GAMMA * mv + ALPHA * av) * rv
        return 0
    lax.fori_loop(0, TPC // 4, _row, 0)

    pltpu.sync_copy(out_v, out_hbm.at[pl.ds(t0, TPC), pl.ds(c0, CPC)])


_ema_update = pl.kernel(
    _body,
    out_type=jax.ShapeDtypeStruct((BOOK, CODE), jnp.float32),
    mesh=plsc.VectorSubcoreMesh(core_axis_name="c", subcore_axis_name="s",
                                num_cores=NC, num_subcores=NS),
    scratch_types=[
        pltpu.VMEM_SHARED((BOOK, CPC), jnp.float32),
        pltpu.VMEM_SHARED((BOOK, L), jnp.float32),
        pltpu.VMEM((TPC,), jnp.int32),
        pltpu.VMEM((NCHUNK, CHUNK), jnp.int32),
        pltpu.VMEM((TPC, CPC), jnp.float32),
        pltpu.VMEM((TPC, CPC), jnp.float32),
        pltpu.VMEM((CHUNK, L), jnp.float32),
        pltpu.VMEM((TPC, CPC), jnp.float32),
        pltpu.VMEM((TPC, L), jnp.float32),
        pltpu.VMEM((TPC, L), jnp.float32),
        pltpu.VMEM((TPC, CPC), jnp.float32),
        pltpu.SemaphoreType.DMA,
        pltpu.SemaphoreType.DMA,
    ],
    compiler_params=pltpu.CompilerParams(use_tc_tiling_on_sc=False),
    name="vq_ema_update_sc",
)


@jax.jit
def kernel(inputs, distances, idx, N, m, codebook):
    del distances, codebook
    n16 = jnp.broadcast_to(N, (BOOK, L))
    return _ema_update(inputs, idx, n16, m)
